# baseline (speedup 1.0000x reference)
import functools

import jax
import jax.numpy as jnp
import numpy as np
from jax import lax
from jax.experimental import pallas as pl
from jax.experimental.pallas import tpu as pltpu
from jax.experimental.pallas import tpu_sc as plsc

_NUM_FIELDS = 26
_EMBED_DIM = 64
_BATCH = 4096
_FIELD_DIMS = [1000] * _NUM_FIELDS
_OFFSETS = np.concatenate(([0], np.cumsum(_FIELD_DIMS)[:-1])).astype(np.int32)
_ROW = []
_COL = []
for _i in range(_NUM_FIELDS - 1):
    for _j in range(_i + 1, _NUM_FIELDS):
        _ROW.append(_i)
        _COL.append(_j)
_NPAIR = len(_ROW)

_PAIRID = np.full((_NUM_FIELDS, _NUM_FIELDS), _NPAIR, dtype=np.int32)
for _k in range(_NPAIR):
    _PAIRID[_COL[_k], _ROW[_k]] = _k

_NC, _NS = 2, 16
_NW = _NC * _NS
_NSEG = 2
_SEG_BATCH = _BATCH // _NSEG
_SEG_ROWS = _SEG_BATCH * _NUM_FIELDS
_ROWS_PER_W = _SEG_ROWS // _NW
_CHUNK = 128
_NCHUNK = _ROWS_PER_W // _CHUNK
_NBUF = 8


def _sc_gather(table, idx3):
    mesh = plsc.VectorSubcoreMesh(core_axis_name="c", subcore_axis_name="s")

    @functools.partial(
        pl.kernel,
        out_type=jax.ShapeDtypeStruct((_SEG_ROWS, _EMBED_DIM), jnp.float32),
        mesh=mesh,
        scratch_types=[
            pltpu.VMEM((_NCHUNK, _CHUNK), jnp.int32),
            pltpu.VMEM((_NBUF, _CHUNK, _EMBED_DIM), jnp.float32),
        ] + [pltpu.SemaphoreType.DMA] * _NBUF,
        compiler_params=pltpu.CompilerParams(use_tc_tiling_on_sc=False),
    )
    def k(table_hbm, idx_hbm, out_hbm, idx_v, rows_v, *sems):
        wid = lax.axis_index("s") * _NC + lax.axis_index("c")
        base = wid * _ROWS_PER_W
        pltpu.sync_copy(idx_hbm.at[wid], idx_v)
        cps = [None] * _NBUF
        for j in range(_NCHUNK + _NBUF - 1):
            if j < _NCHUNK:
                b = j % _NBUF
                cps[b] = pltpu.async_copy(
                    table_hbm.at[idx_v.at[j]], rows_v.at[b], sems[b])
            if j >= _NBUF - 1:
                p = j - (_NBUF - 1)
                pb = p % _NBUF
                cps[pb].wait()
                pltpu.sync_copy(
                    rows_v.at[pb],
                    out_hbm.at[pl.ds(base + p * _CHUNK, _CHUNK)])

    return k(table, idx3)


_BB = 512


_D = _NUM_FIELDS * _EMBED_DIM
_SPLIT = 896


def _kb_body(m_ref, p1t_ref, p1_ref, kb_ref):
    t1 = jnp.dot(p1t_ref[...], m_ref[...],
                 preferred_element_type=jnp.float32)
    me = jnp.dot(t1.astype(jnp.bfloat16), p1_ref[...],
                 preferred_element_type=jnp.float32)
    r = lax.broadcasted_iota(jnp.int32, (_D, _D), 0)
    c = lax.broadcasted_iota(jnp.int32, (_D, _D), 1)
    mask = (r % _EMBED_DIM) == (c % _EMBED_DIM)
    kb_ref[...] = jnp.where(mask, me, 0.0).astype(jnp.bfloat16)


def _kb_build(m, p1t, p1):
    return pl.pallas_call(
        _kb_body,
        out_shape=jax.ShapeDtypeStruct((_D, _D), jnp.bfloat16),
    )(m, p1t, p1)


def _tc_body(e_ref, kb_ref, wl_ref, b_ref, o_ref):
    e = e_ref[...]
    eb = e.astype(jnp.bfloat16)
    g1 = jnp.dot(eb, kb_ref[:, : _SPLIT],
                 preferred_element_type=jnp.float32)
    g2 = jnp.dot(eb[:, _SPLIT:], kb_ref[_SPLIT:, _SPLIT:],
                 preferred_element_type=jnp.float32)
    g = jnp.concatenate([g1, g2], axis=1)
    t = e * (g + wl_ref[...])
    logit = jnp.sum(t, axis=1) + b_ref[0]
    o_ref[...] = jax.nn.sigmoid(logit)


def _tc_fwfm(embed, kb, w_lin, b_int):
    grid = (_SEG_BATCH // _BB,)
    return pl.pallas_call(
        _tc_body,
        grid=grid,
        in_specs=[
            pl.BlockSpec((_BB, _D), lambda g: (g, 0)),
            pl.BlockSpec((_D, _D), lambda g: (0, 0)),
            pl.BlockSpec((1, _D), lambda g: (0, 0)),
            pl.BlockSpec(memory_space=pltpu.SMEM),
        ],
        out_specs=pl.BlockSpec((_BB,), lambda g: (g,)),
        out_shape=jax.ShapeDtypeStruct((_SEG_BATCH,), jnp.float32),
    )(embed, kb, w_lin, b_int)


def kernel(x, table, W_int, b_int, W_lin):
    idx = (x + jnp.asarray(_OFFSETS)[None, :]).reshape(-1)

    w_pad = jnp.concatenate(
        [W_int[0, :], jnp.zeros((1,), jnp.float32)]).astype(jnp.bfloat16)
    m = jnp.take(w_pad, jnp.asarray(_PAIRID))
    p1np = np.zeros((_NUM_FIELDS, _D), dtype=np.float32)
    p1np[np.arange(_D) // _EMBED_DIM, np.arange(_D)] = 1.0
    p1 = jnp.asarray(p1np, dtype=jnp.bfloat16)
    p1t = jnp.asarray(p1np.T.copy(), dtype=jnp.bfloat16)
    kb = _kb_build(m, p1t, p1)

    outs = []
    for s in range(_NSEG):
        idx3 = lax.slice_in_dim(idx, s * _SEG_ROWS, (s + 1) * _SEG_ROWS
                                ).reshape(_NW, _NCHUNK, _CHUNK)
        emb = _sc_gather(table, idx3)
        emb = emb.reshape(_SEG_BATCH, _NUM_FIELDS * _EMBED_DIM)
        outs.append(_tc_fwfm(emb, kb, W_lin, b_int))
    return jnp.concatenate(outs)


# speedup vs baseline: 1.0502x; 1.0502x over previous
import functools

import jax
import jax.numpy as jnp
import numpy as np
from jax import lax
from jax.experimental import pallas as pl
from jax.experimental.pallas import tpu as pltpu
from jax.experimental.pallas import tpu_sc as plsc

_NUM_FIELDS = 26
_EMBED_DIM = 64
_BATCH = 4096
_FIELD_DIMS = [1000] * _NUM_FIELDS
_OFFSETS = np.concatenate(([0], np.cumsum(_FIELD_DIMS)[:-1])).astype(np.int32)
_ROW = []
_COL = []
for _i in range(_NUM_FIELDS - 1):
    for _j in range(_i + 1, _NUM_FIELDS):
        _ROW.append(_i)
        _COL.append(_j)
_NPAIR = len(_ROW)

_PAIRID = np.full((_NUM_FIELDS, _NUM_FIELDS), _NPAIR, dtype=np.int32)
for _k in range(_NPAIR):
    _PAIRID[_COL[_k], _ROW[_k]] = _k

_NC, _NS = 2, 16
_NW = _NC * _NS
_NSEG = 1
_SEG_BATCH = _BATCH // _NSEG
_SEG_ROWS = _SEG_BATCH * _NUM_FIELDS
_ROWS_PER_W = _SEG_ROWS // _NW
_CHUNK = 128
_NCHUNK = _ROWS_PER_W // _CHUNK
_NBUF = 8


def _sc_gather(table, idx3):
    mesh = plsc.VectorSubcoreMesh(core_axis_name="c", subcore_axis_name="s")

    @functools.partial(
        pl.kernel,
        out_type=jax.ShapeDtypeStruct((_SEG_ROWS, _EMBED_DIM), jnp.float32),
        mesh=mesh,
        scratch_types=[
            pltpu.VMEM((_NCHUNK, _CHUNK), jnp.int32),
            pltpu.VMEM((_NBUF, _CHUNK, _EMBED_DIM), jnp.float32),
        ] + [pltpu.SemaphoreType.DMA] * _NBUF,
        compiler_params=pltpu.CompilerParams(use_tc_tiling_on_sc=False),
    )
    def k(table_hbm, idx_hbm, out_hbm, idx_v, rows_v, *sems):
        wid = lax.axis_index("s") * _NC + lax.axis_index("c")
        base = wid * _ROWS_PER_W
        pltpu.sync_copy(idx_hbm.at[wid], idx_v)
        cps = [None] * _NBUF
        for j in range(_NCHUNK + _NBUF - 1):
            if j < _NCHUNK:
                b = j % _NBUF
                cps[b] = pltpu.async_copy(
                    table_hbm.at[idx_v.at[j]], rows_v.at[b], sems[b])
            if j >= _NBUF - 1:
                p = j - (_NBUF - 1)
                pb = p % _NBUF
                cps[pb].wait()
                pltpu.sync_copy(
                    rows_v.at[pb],
                    out_hbm.at[pl.ds(base + p * _CHUNK, _CHUNK)])

    return k(table, idx3)


_BB = 512


_D = _NUM_FIELDS * _EMBED_DIM
_TRI = [(0, 384, 0), (384, 768, 384), (768, 1152, 768), (1152, 1664, 1152)]


def _kb_body(m_ref, p1t_ref, p1_ref, kb_ref):
    t1 = jnp.dot(p1t_ref[...], m_ref[...],
                 preferred_element_type=jnp.float32)
    me = jnp.dot(t1.astype(jnp.bfloat16), p1_ref[...],
                 preferred_element_type=jnp.float32)
    r = lax.broadcasted_iota(jnp.int32, (_D, _D), 0)
    c = lax.broadcasted_iota(jnp.int32, (_D, _D), 1)
    mask = (r % _EMBED_DIM) == (c % _EMBED_DIM)
    kb_ref[...] = jnp.where(mask, me, 0.0).astype(jnp.bfloat16)


def _kb_build(m, p1t, p1):
    return pl.pallas_call(
        _kb_body,
        out_shape=jax.ShapeDtypeStruct((_D, _D), jnp.bfloat16),
    )(m, p1t, p1)


def _tc_body(e_ref, kb_ref, wl_ref, b_ref, o_ref):
    e = e_ref[...]
    eb = e.astype(jnp.bfloat16)
    g = jnp.concatenate(
        [jnp.dot(eb[:, rs:], kb_ref[rs:, cs:ce],
                 preferred_element_type=jnp.float32)
         for cs, ce, rs in _TRI], axis=1)
    t = e * (g + wl_ref[...])
    logit = jnp.sum(t, axis=1) + b_ref[0]
    o_ref[...] = jax.nn.sigmoid(logit)


def _tc_fwfm(embed, kb, w_lin, b_int):
    grid = (_SEG_BATCH // _BB,)
    return pl.pallas_call(
        _tc_body,
        grid=grid,
        in_specs=[
            pl.BlockSpec((_BB, _D), lambda g: (g, 0)),
            pl.BlockSpec((_D, _D), lambda g: (0, 0)),
            pl.BlockSpec((1, _D), lambda g: (0, 0)),
            pl.BlockSpec(memory_space=pltpu.SMEM),
        ],
        out_specs=pl.BlockSpec((_BB,), lambda g: (g,)),
        out_shape=jax.ShapeDtypeStruct((_SEG_BATCH,), jnp.float32),
    )(embed, kb, w_lin, b_int)


def kernel(x, table, W_int, b_int, W_lin):
    idx = (x + jnp.asarray(_OFFSETS)[None, :]).reshape(-1)

    w_pad = jnp.concatenate(
        [W_int[0, :], jnp.zeros((1,), jnp.float32)]).astype(jnp.bfloat16)
    m = jnp.take(w_pad, jnp.asarray(_PAIRID))
    p1np = np.zeros((_NUM_FIELDS, _D), dtype=np.float32)
    p1np[np.arange(_D) // _EMBED_DIM, np.arange(_D)] = 1.0
    p1 = jnp.asarray(p1np, dtype=jnp.bfloat16)
    p1t = jnp.asarray(p1np.T.copy(), dtype=jnp.bfloat16)
    kb = _kb_build(m, p1t, p1)

    outs = []
    for s in range(_NSEG):
        idx3 = lax.slice_in_dim(idx, s * _SEG_ROWS, (s + 1) * _SEG_ROWS
                                ).reshape(_NW, _NCHUNK, _CHUNK)
        emb = _sc_gather(table, idx3)
        emb = emb.reshape(_SEG_BATCH, _NUM_FIELDS * _EMBED_DIM)
        outs.append(_tc_fwfm(emb, kb, W_lin, b_int))
    return jnp.concatenate(outs)


# speedup vs baseline: 1.0601x; 1.0094x over previous
import functools

import jax
import jax.numpy as jnp
import numpy as np
from jax import lax
from jax.experimental import pallas as pl
from jax.experimental.pallas import tpu as pltpu
from jax.experimental.pallas import tpu_sc as plsc

_NUM_FIELDS = 26
_EMBED_DIM = 64
_BATCH = 4096
_FIELD_DIMS = [1000] * _NUM_FIELDS
_OFFSETS = np.concatenate(([0], np.cumsum(_FIELD_DIMS)[:-1])).astype(np.int32)
_ROW = []
_COL = []
for _i in range(_NUM_FIELDS - 1):
    for _j in range(_i + 1, _NUM_FIELDS):
        _ROW.append(_i)
        _COL.append(_j)
_NPAIR = len(_ROW)

_PAIRID = np.full((_NUM_FIELDS, _NUM_FIELDS), _NPAIR, dtype=np.int32)
for _k in range(_NPAIR):
    _PAIRID[_COL[_k], _ROW[_k]] = _k

_NC, _NS = 2, 16
_NW = _NC * _NS
_NSEG = 1
_SEG_BATCH = _BATCH // _NSEG
_SEG_ROWS = _SEG_BATCH * _NUM_FIELDS
_ROWS_PER_W = _SEG_ROWS // _NW
_CHUNK = 128
_NCHUNK = _ROWS_PER_W // _CHUNK
_NBUF = 13


def _sc_gather(table, idx3):
    mesh = plsc.VectorSubcoreMesh(core_axis_name="c", subcore_axis_name="s")

    @functools.partial(
        pl.kernel,
        out_type=jax.ShapeDtypeStruct((_SEG_ROWS, _EMBED_DIM), jnp.float32),
        mesh=mesh,
        scratch_types=[
            pltpu.VMEM((_NCHUNK, _CHUNK), jnp.int32),
            pltpu.VMEM((_NBUF, _CHUNK, _EMBED_DIM), jnp.float32),
        ] + [pltpu.SemaphoreType.DMA] * _NBUF,
        compiler_params=pltpu.CompilerParams(use_tc_tiling_on_sc=False),
    )
    def k(table_hbm, idx_hbm, out_hbm, idx_v, rows_v, *sems):
        wid = lax.axis_index("s") * _NC + lax.axis_index("c")
        base = wid * _ROWS_PER_W
        pltpu.sync_copy(idx_hbm.at[wid], idx_v)
        cps = [None] * _NBUF
        for j in range(_NCHUNK + _NBUF - 1):
            if j < _NCHUNK:
                b = j % _NBUF
                cps[b] = pltpu.async_copy(
                    table_hbm.at[idx_v.at[j]], rows_v.at[b], sems[b])
            if j >= _NBUF - 1:
                p = j - (_NBUF - 1)
                pb = p % _NBUF
                cps[pb].wait()
                pltpu.sync_copy(
                    rows_v.at[pb],
                    out_hbm.at[pl.ds(base + p * _CHUNK, _CHUNK)])

    return k(table, idx3)


_BB = 512


_D = _NUM_FIELDS * _EMBED_DIM
_TRI = [(0, 384, 0), (384, 768, 384), (768, 1152, 768), (1152, 1664, 1152)]


def _kb_body(m_ref, p1t_ref, p1_ref, kb_ref):
    t1 = jnp.dot(p1t_ref[...], m_ref[...],
                 preferred_element_type=jnp.float32)
    me = jnp.dot(t1.astype(jnp.bfloat16), p1_ref[...],
                 preferred_element_type=jnp.float32)
    r = lax.broadcasted_iota(jnp.int32, (_D, _D), 0)
    c = lax.broadcasted_iota(jnp.int32, (_D, _D), 1)
    mask = (r % _EMBED_DIM) == (c % _EMBED_DIM)
    kb_ref[...] = jnp.where(mask, me, 0.0).astype(jnp.bfloat16)


def _kb_build(m, p1t, p1):
    return pl.pallas_call(
        _kb_body,
        out_shape=jax.ShapeDtypeStruct((_D, _D), jnp.bfloat16),
    )(m, p1t, p1)


def _tc_body(e_ref, kb_ref, wl_ref, b_ref, o_ref):
    e = e_ref[...]
    eb = e.astype(jnp.bfloat16)
    g = jnp.concatenate(
        [jnp.dot(eb[:, rs:], kb_ref[rs:, cs:ce],
                 preferred_element_type=jnp.float32)
         for cs, ce, rs in _TRI], axis=1)
    t = e * (g + wl_ref[...])
    logit = jnp.sum(t, axis=1) + b_ref[0]
    o_ref[...] = jax.nn.sigmoid(logit)


def _tc_fwfm(embed, kb, w_lin, b_int):
    grid = (_SEG_BATCH // _BB,)
    return pl.pallas_call(
        _tc_body,
        grid=grid,
        in_specs=[
            pl.BlockSpec((_BB, _D), lambda g: (g, 0)),
            pl.BlockSpec((_D, _D), lambda g: (0, 0)),
            pl.BlockSpec((1, _D), lambda g: (0, 0)),
            pl.BlockSpec(memory_space=pltpu.SMEM),
        ],
        out_specs=pl.BlockSpec((_BB,), lambda g: (g,)),
        out_shape=jax.ShapeDtypeStruct((_SEG_BATCH,), jnp.float32),
    )(embed, kb, w_lin, b_int)


def kernel(x, table, W_int, b_int, W_lin):
    idx = (x + jnp.asarray(_OFFSETS)[None, :]).reshape(-1)

    w_pad = jnp.concatenate(
        [W_int[0, :], jnp.zeros((1,), jnp.float32)]).astype(jnp.bfloat16)
    m = jnp.take(w_pad, jnp.asarray(_PAIRID))
    p1np = np.zeros((_NUM_FIELDS, _D), dtype=np.float32)
    p1np[np.arange(_D) // _EMBED_DIM, np.arange(_D)] = 1.0
    p1 = jnp.asarray(p1np, dtype=jnp.bfloat16)
    p1t = jnp.asarray(p1np.T.copy(), dtype=jnp.bfloat16)
    kb = _kb_build(m, p1t, p1)

    outs = []
    for s in range(_NSEG):
        idx3 = lax.slice_in_dim(idx, s * _SEG_ROWS, (s + 1) * _SEG_ROWS
                                ).reshape(_NW, _NCHUNK, _CHUNK)
        emb = _sc_gather(table, idx3)
        emb = emb.reshape(_SEG_BATCH, _NUM_FIELDS * _EMBED_DIM)
        outs.append(_tc_fwfm(emb, kb, W_lin, b_int))
    return jnp.concatenate(outs)
